# speedup vs baseline: 6.5944x; 6.5944x over previous
import functools

import jax
import jax.numpy as jnp
from jax import lax
from jax.experimental import pallas as pl
from jax.experimental.pallas import tpu as pltpu
from jax.experimental.pallas import tpu_sc as plsc

D = 128
NC, NS = 2, 16
NW = NC * NS


@functools.lru_cache(maxsize=None)
def _gather_call(B, CP):
    rows_total = B * CP
    b_per_w = B // NW
    c_per_w = rows_total // NW
    CH = b_per_w
    n_chunks = c_per_w // CH
    mesh = plsc.VectorSubcoreMesh(core_axis_name="c", subcore_axis_name="s")

    @functools.partial(
        pl.kernel,
        mesh=mesh,
        out_type=[
            jax.ShapeDtypeStruct((B, D), jnp.float32),
            jax.ShapeDtypeStruct((rows_total, D), jnp.float32),
        ],
        scratch_types=[
            pltpu.VMEM((b_per_w,), jnp.int32),
            pltpu.VMEM((CH,), jnp.int32),
            pltpu.VMEM((CH, D), jnp.float32),
            pltpu.SemaphoreType.DMA,
        ],
    )
    def gather_kernel(wemb_hbm, cemb_hbm, widx_hbm, cidx_hbm,
                      wout_hbm, cout_hbm, widx_v, cidx_v, rows_v, sem):
        wid = lax.axis_index("s") * NC + lax.axis_index("c")
        wbase = wid * b_per_w
        pltpu.sync_copy(widx_hbm.at[pl.ds(wbase, b_per_w)], widx_v)
        pltpu.async_copy(wemb_hbm.at[widx_v], rows_v, sem).wait()
        pltpu.sync_copy(rows_v, wout_hbm.at[pl.ds(wbase, b_per_w)])

        cbase = wid * c_per_w

        @pl.loop(0, n_chunks)
        def _(i):
            off = cbase + i * CH
            pltpu.sync_copy(cidx_hbm.at[pl.ds(off, CH)], cidx_v)
            pltpu.async_copy(cemb_hbm.at[cidx_v], rows_v, sem).wait()
            pltpu.sync_copy(rows_v, cout_hbm.at[pl.ds(off, CH)])

    return gather_kernel


@functools.lru_cache(maxsize=None)
def _dot_call(B, CP, P, interpret=False):
    BB = 512
    grid = (B // BB,)

    def body(w_ref, c_ref, p_ref, n_ref):
        w = w_ref[...]
        c = c_ref[...].reshape(BB, CP, D)
        logits = jnp.sum(c * w[:, None, :], axis=-1)
        s = jax.nn.sigmoid(logits)
        p_ref[...] = s[:, :P]
        n_ref[...] = s[:, P:]

    return pl.pallas_call(
        body,
        grid=grid,
        in_specs=[
            pl.BlockSpec((BB, D), lambda i: (i, 0)),
            pl.BlockSpec((BB * CP, D), lambda i: (i, 0)),
        ],
        out_specs=[
            pl.BlockSpec((BB, P), lambda i: (i, 0)),
            pl.BlockSpec((BB, CP - P), lambda i: (i, 0)),
        ],
        out_shape=[
            jax.ShapeDtypeStruct((B, P), jnp.float32),
            jax.ShapeDtypeStruct((B, CP - P), jnp.float32),
        ],
        interpret=interpret,
    )


def kernel(words, positive_contexts, negative_contexts, word_emb, context_emb):
    B = words.shape[0]
    P = positive_contexts.shape[1]
    N = negative_contexts.shape[1]
    CP = P + N
    cidx = jnp.concatenate([positive_contexts, negative_contexts],
                           axis=1).reshape(B * CP)
    wrows, crows = _gather_call(B, CP)(word_emb, context_emb, words, cidx)
    pos, neg = _dot_call(B, CP, P)(wrows, crows)
    return pos, neg
